# Initial kernel scaffold; baseline (speedup 1.0000x reference)
import jax
import jax.numpy as jnp
from jax.experimental import pallas as pl


def kernel(gradient, block_edge_index, edge_index, edge_weight, step_size):
    raise NotImplementedError("write your pallas kernel here")


# speedup vs baseline: 1.0000x; 1.0000x over previous
import jax
import jax.numpy as jnp
from jax.experimental import pallas as pl

NUM_NODES = 50000
STEP_SIZE = 2048


def kernel(gradient, block_edge_index, edge_index, edge_weight, step_size):
    topv, topi = jax.lax.top_k(gradient, STEP_SIZE)
    flip_ei = block_edge_index[:, topi]
    flip_ei_full = jnp.concatenate([flip_ei, flip_ei[::-1, :]], axis=1)
    flip_w_full = jnp.ones((2 * STEP_SIZE,), dtype=jnp.float32) * (step_size // STEP_SIZE)
    comb_ei = jnp.concatenate([edge_index, flip_ei_full], axis=1)
    comb_w = jnp.concatenate([edge_weight, flip_w_full])
    ids = comb_ei[0] * NUM_NODES + comb_ei[1]
    order = jnp.argsort(ids)
    ids_s = ids[order]
    w_s = comb_w[order]
    src_s = comb_ei[0][order]
    dst_s = comb_ei[1][order]
    new_run = jnp.concatenate([jnp.array([True]), ids_s[1:] != ids_s[:-1]])
    run_id = jnp.cumsum(new_run) - 1
    total = ids.shape[0]
    summed = jax.ops.segment_sum(w_s, run_id, num_segments=total)
    rep_src = jax.ops.segment_max(src_s, run_id, num_segments=total)
    rep_dst = jax.ops.segment_max(dst_s, run_id, num_segments=total)
    is_one = jnp.isclose(summed, 1.0)
    final_w = jnp.where(is_one, summed, 0.0)
    n_pos = jnp.sum(gradient > 0)
    return final_w, rep_src, rep_dst, topv, flip_ei, n_pos


# speedup vs baseline: 2.0088x; 2.0088x over previous
import jax
import jax.numpy as jnp
from jax import lax
from jax.experimental import pallas as pl
from jax.experimental.pallas import tpu as pltpu

NUM_NODES = 50000
STEP_SIZE = 2048
N_GRAD = 1000000
GPAD = 1 << 20
N_EDGES = 1600000
TOT = N_EDGES + 2 * STEP_SIZE
LOGN = 21
SPAD = 1 << LOGN
RB = 512
NBLK = SPAD // (RB * 128)
LOGM = 16

I32 = jnp.int32
F32 = jnp.float32


def _keys_body(g_ref, key_ref, npos_ref):
    x = g_ref[...]
    b = lax.bitcast_convert_type(x, I32)
    key = b ^ jnp.bitwise_and(jnp.right_shift(b, 31), I32(0x7FFFFFFF))
    key_ref[...] = key
    cnt = jnp.sum(jnp.where(key > I32(0), F32(1.0), F32(0.0)))
    npos_ref[0, 0, 0] = cnt


def _make_keys(gpad2d):
    return pl.pallas_call(
        _keys_body,
        grid=(8,),
        in_specs=[pl.BlockSpec((128, 1024), lambda i: (i, I32(0)))],
        out_specs=[
            pl.BlockSpec((128, 1024), lambda i: (i, I32(0))),
            pl.BlockSpec((1, 1, 1), lambda i: (i, I32(0), I32(0)), memory_space=pltpu.SMEM),
        ],
        out_shape=[
            jax.ShapeDtypeStruct((1024, 1024), I32),
            jax.ShapeDtypeStruct((8, 1, 1), F32),
        ],
    )(gpad2d)


def _cmpex(key, w, amt, jbits, coord, dir_asc, axis):
    lo = jnp.bitwise_and(lax.shift_right_logical(coord, jbits), I32(1)) == I32(0)
    sz = I32(key.shape[axis])
    pm = pltpu.roll(key, sz - amt, axis=axis)
    pp = pltpu.roll(key, amt, axis=axis)
    p = jnp.where(lo, pm, pp)
    wm = pltpu.roll(w, sz - amt, axis=axis)
    wp = pltpu.roll(w, amt, axis=axis)
    pw = jnp.where(lo, wm, wp)
    take_min = lo == dir_asc
    cmp_ = (key < p) | ((key == p) & lo)
    cha = cmp_ == take_min
    return jnp.where(cha, key, p), jnp.where(cha, w, pw)


def _run_level_stages(key, w, k, jhi, dir_asc, lane, sub):
    if jhi >= 7:
        def sbody(jkw):
            j, kk, ww = jkw
            amt = lax.shift_left(I32(1), j - I32(7))
            kk, ww = _cmpex(kk, ww, amt, j - I32(7), sub, dir_asc, 0)
            return (j - I32(1), kk, ww)
        _, key, w = lax.while_loop(lambda jkw: jkw[0] >= I32(7), sbody,
                                   (I32(jhi), key, w))
    jl = min(jhi, 6)
    def lbody(jkw):
        j, kk, ww = jkw
        amt = lax.shift_left(I32(1), j)
        kk, ww = _cmpex(kk, ww, amt, j, lane, dir_asc, 1)
        return (j - I32(1), kk, ww)
    _, key, w = lax.while_loop(lambda jkw: jkw[0] >= I32(0), lbody,
                               (I32(jl), key, w))
    return key, w


def _local_sort_body(k_ref, w_ref, ok_ref, ow_ref):
    b = pl.program_id(0)
    key = k_ref[...]
    w = w_ref[...]
    shape = key.shape
    lane = lax.broadcasted_iota(I32, shape, 1)
    sub = lax.broadcasted_iota(I32, shape, 0)
    for k in range(1, LOGM + 1):
        if k <= 6:
            dirbit = jnp.bitwise_and(lax.shift_right_logical(lane, I32(k)), I32(1))
        elif k < LOGM:
            dirbit = jnp.bitwise_and(lax.shift_right_logical(sub, I32(k - 7)), I32(1))
        else:
            dirbit = jnp.broadcast_to(jnp.bitwise_and(b, I32(1)), shape)
        dir_asc = dirbit == I32(0)
        key, w = _run_level_stages(key, w, k, k - 1, dir_asc, lane, sub)
    ok_ref[...] = key
    ow_ref[...] = w


def _finish_body_factory(k):
    def _finish_body(k_ref, w_ref, ok_ref, ow_ref):
        b = pl.program_id(0)
        key = k_ref[...]
        w = w_ref[...]
        shape = key.shape
        lane = lax.broadcasted_iota(I32, shape, 1)
        sub = lax.broadcasted_iota(I32, shape, 0)
        dirbit = jnp.bitwise_and(lax.shift_right_logical(b, I32(k - LOGM)), I32(1))
        dir_asc = jnp.broadcast_to(dirbit, shape) == I32(0)
        key, w = _run_level_stages(key, w, k, 15, dir_asc, lane, sub)
        ok_ref[...] = key
        ow_ref[...] = w
    return _finish_body


def _pair_body_factory(k, j):
    def _pair_body(k_ref, w_ref, ok_ref, ow_ref):
        g = pl.program_id(0)
        a = k_ref[0, 0]
        bb = k_ref[0, 1]
        wa = w_ref[0, 0]
        wb = w_ref[0, 1]
        dirbit = jnp.bitwise_and(
            lax.shift_right_logical(g, I32(k - j - 1)), I32(1))
        dir_asc = jnp.broadcast_to(dirbit, a.shape) == I32(0)
        cha = ((a < bb) | (a == bb)) == dir_asc
        ok_ref[0, 0] = jnp.where(cha, a, bb)
        ok_ref[0, 1] = jnp.where(cha, bb, a)
        ow_ref[0, 0] = jnp.where(cha, wa, wb)
        ow_ref[0, 1] = jnp.where(cha, wb, wa)
    return _pair_body


def _bitonic_sort(key2d, w2d):
    blk = pl.BlockSpec((RB, 128), lambda i: (i, I32(0)))
    shp = [jax.ShapeDtypeStruct((SPAD // 128, 128), I32),
           jax.ShapeDtypeStruct((SPAD // 128, 128), F32)]
    key2d, w2d = pl.pallas_call(
        _local_sort_body, grid=(NBLK,),
        in_specs=[blk, blk], out_specs=[blk, blk], out_shape=shp,
    )(key2d, w2d)
    nrows = SPAD // 128
    for k in range(LOGM + 1, LOGN + 1):
        for j in range(k - 1, LOGM - 1, -1):
            db = 1 << (j - LOGM)
            half = db * RB
            G = nrows // (2 * half)
            k4 = key2d.reshape(G, 2, half, 128)
            w4 = w2d.reshape(G, 2, half, 128)
            pblk = pl.BlockSpec((1, 2, RB, 128), lambda g, i: (g, I32(0), i, I32(0)))
            k4, w4 = pl.pallas_call(
                _pair_body_factory(k, j), grid=(G, db),
                in_specs=[pblk, pblk], out_specs=[pblk, pblk],
                out_shape=[jax.ShapeDtypeStruct((G, 2, half, 128), I32),
                           jax.ShapeDtypeStruct((G, 2, half, 128), F32)],
            )(k4, w4)
            key2d = k4.reshape(nrows, 128)
            w2d = w4.reshape(nrows, 128)
        key2d, w2d = pl.pallas_call(
            _finish_body_factory(k), grid=(NBLK,),
            in_specs=[blk, blk], out_specs=[blk, blk], out_shape=shp,
        )(key2d, w2d)
    return key2d, w2d


def kernel(gradient, block_edge_index, edge_index, edge_weight, step_size):
    gpad = jnp.concatenate(
        [gradient, jnp.full((GPAD - N_GRAD,), -jnp.inf, F32)]).reshape(1024, 1024)
    keys2d, npos_part = _make_keys(gpad)
    key = keys2d.reshape(GPAD)
    n_pos = jnp.sum(npos_part).astype(jnp.int64)

    topv, topi = lax.top_k(gradient, STEP_SIZE)

    flip_ei = block_edge_index[:, topi]
    fsrc = flip_ei[0].astype(jnp.uint32)
    fdst = flip_ei[1].astype(jnp.uint32)
    fw = jnp.asarray(step_size // STEP_SIZE).astype(F32)
    esrc = edge_index[0].astype(jnp.uint32)
    edst = edge_index[1].astype(jnp.uint32)
    NN = jnp.uint32(NUM_NODES)
    ids = jnp.concatenate([esrc * NN + edst, fsrc * NN + fdst, fdst * NN + fsrc])
    w = jnp.concatenate([edge_weight.astype(F32),
                         jnp.full((2 * STEP_SIZE,), 1.0, F32) * fw])

    skey = (ids ^ jnp.uint32(0x80000000)).astype(I32)
    skey_p = jnp.concatenate(
        [skey, jnp.full((SPAD - TOT,), I32(0x7FFFFFFF))]).reshape(SPAD // 128, 128)
    w_p = jnp.concatenate(
        [w, jnp.zeros((SPAD - TOT,), F32)]).reshape(SPAD // 128, 128)
    skey_s2d, w_s2d = _bitonic_sort(skey_p, w_p)
    ids_s = (skey_s2d.reshape(SPAD).astype(jnp.uint32)
             ^ jnp.uint32(0x80000000))[:TOT]
    w_s = w_s2d.reshape(SPAD)[:TOT]

    newrun = jnp.concatenate([jnp.array([True]), ids_s[1:] != ids_s[:-1]])
    S = jnp.cumsum(w_s)
    run_id = jnp.cumsum(newrun) - 1
    nruns = run_id[-1] + 1
    pos = jnp.arange(TOT)
    run_starts = jnp.full((TOT,), TOT, jnp.int64).at[
        jnp.where(newrun, run_id, TOT)].min(pos, mode='drop')
    valid = pos < nruns
    rs = jnp.where(valid, run_starts, 0)
    next_start = jnp.where(pos + 1 < nruns, jnp.roll(run_starts, -1), TOT)
    end_idx = jnp.where(valid, next_start - 1, 0)
    summed = jnp.where(valid, S[end_idx] - jnp.where(rs > 0, S[rs - 1], 0.0), 0.0
                       ).astype(F32)
    id_k = ids_s[rs]
    PAD = jnp.iinfo(jnp.int64).min
    rep_src = jnp.where(valid, (id_k // NUM_NODES).astype(jnp.int64), PAD)
    rep_dst = jnp.where(valid, (id_k % NUM_NODES).astype(jnp.int64), PAD)
    final_w = jnp.where(jnp.isclose(summed, 1.0), summed, 0.0)
    return final_w, rep_src, rep_dst, topv, flip_ei, n_pos


# speedup vs baseline: 40.8717x; 20.3460x over previous
import jax
import jax.numpy as jnp
from jax import lax
from jax.experimental import pallas as pl
from jax.experimental.pallas import tpu as pltpu

NUM_NODES = 50000
STEP_SIZE = 2048
N_GRAD = 1000000
GPAD = 1 << 20
N_EDGES = 1600000
TOT = N_EDGES + 2 * STEP_SIZE
LOGN = 21
SPAD = 1 << LOGN
RB = 512
NBLK = SPAD // (RB * 128)
LOGM = 16

I32 = jnp.int32
F32 = jnp.float32


def _keys_body(g_ref, key_ref, npos_ref):
    x = g_ref[...]
    b = lax.bitcast_convert_type(x, I32)
    key = b ^ jnp.bitwise_and(jnp.right_shift(b, 31), I32(0x7FFFFFFF))
    key_ref[...] = key
    cnt = jnp.sum(jnp.where(key > I32(0), F32(1.0), F32(0.0)))
    npos_ref[0, 0, 0] = cnt


def _make_keys(gpad2d):
    return pl.pallas_call(
        _keys_body,
        grid=(8,),
        in_specs=[pl.BlockSpec((128, 1024), lambda i: (i, I32(0)))],
        out_specs=[
            pl.BlockSpec((128, 1024), lambda i: (i, I32(0))),
            pl.BlockSpec((1, 1, 1), lambda i: (i, I32(0), I32(0)), memory_space=pltpu.SMEM),
        ],
        out_shape=[
            jax.ShapeDtypeStruct((1024, 1024), I32),
            jax.ShapeDtypeStruct((8, 1, 1), F32),
        ],
    )(gpad2d)


def _cmpex(key, w, amt, jbits, coord, dir_asc, axis):
    lo = jnp.bitwise_and(lax.shift_right_logical(coord, jbits), I32(1)) == I32(0)
    sz = I32(key.shape[axis])
    pm = pltpu.roll(key, sz - amt, axis=axis)
    pp = pltpu.roll(key, amt, axis=axis)
    p = jnp.where(lo, pm, pp)
    wm = pltpu.roll(w, sz - amt, axis=axis)
    wp = pltpu.roll(w, amt, axis=axis)
    pw = jnp.where(lo, wm, wp)
    take_min = lo == dir_asc
    cmp_ = (key < p) | ((key == p) & lo)
    cha = cmp_ == take_min
    return jnp.where(cha, key, p), jnp.where(cha, w, pw)


def _run_level_stages(key, w, k, jhi, dir_asc, lane, sub):
    if jhi >= 7:
        def sbody(jkw):
            j, kk, ww = jkw
            amt = lax.shift_left(I32(1), j - I32(7))
            kk, ww = _cmpex(kk, ww, amt, j - I32(7), sub, dir_asc, 0)
            return (j - I32(1), kk, ww)
        _, key, w = lax.while_loop(lambda jkw: jkw[0] >= I32(7), sbody,
                                   (I32(jhi), key, w))
    jl = min(jhi, 6)
    def lbody(jkw):
        j, kk, ww = jkw
        amt = lax.shift_left(I32(1), j)
        kk, ww = _cmpex(kk, ww, amt, j, lane, dir_asc, 1)
        return (j - I32(1), kk, ww)
    _, key, w = lax.while_loop(lambda jkw: jkw[0] >= I32(0), lbody,
                               (I32(jl), key, w))
    return key, w


def _local_sort_body(k_ref, w_ref, ok_ref, ow_ref):
    b = pl.program_id(0)
    key = k_ref[...]
    w = w_ref[...]
    shape = key.shape
    lane = lax.broadcasted_iota(I32, shape, 1)
    sub = lax.broadcasted_iota(I32, shape, 0)
    for k in range(1, LOGM + 1):
        if k <= 6:
            dirbit = jnp.bitwise_and(lax.shift_right_logical(lane, I32(k)), I32(1))
        elif k < LOGM:
            dirbit = jnp.bitwise_and(lax.shift_right_logical(sub, I32(k - 7)), I32(1))
        else:
            dirbit = jnp.broadcast_to(jnp.bitwise_and(b, I32(1)), shape)
        dir_asc = dirbit == I32(0)
        key, w = _run_level_stages(key, w, k, k - 1, dir_asc, lane, sub)
    ok_ref[...] = key
    ow_ref[...] = w


def _finish_body_factory(k):
    def _finish_body(k_ref, w_ref, ok_ref, ow_ref):
        b = pl.program_id(0)
        key = k_ref[...]
        w = w_ref[...]
        shape = key.shape
        lane = lax.broadcasted_iota(I32, shape, 1)
        sub = lax.broadcasted_iota(I32, shape, 0)
        dirbit = jnp.bitwise_and(lax.shift_right_logical(b, I32(k - LOGM)), I32(1))
        dir_asc = jnp.broadcast_to(dirbit, shape) == I32(0)
        key, w = _run_level_stages(key, w, k, 15, dir_asc, lane, sub)
        ok_ref[...] = key
        ow_ref[...] = w
    return _finish_body


def _pair_body_factory(k, j):
    def _pair_body(k_ref, w_ref, ok_ref, ow_ref):
        g = pl.program_id(0)
        a = k_ref[0, 0]
        bb = k_ref[0, 1]
        wa = w_ref[0, 0]
        wb = w_ref[0, 1]
        dirbit = jnp.bitwise_and(
            lax.shift_right_logical(g, I32(k - j - 1)), I32(1))
        dir_asc = jnp.broadcast_to(dirbit, a.shape) == I32(0)
        cha = ((a < bb) | (a == bb)) == dir_asc
        ok_ref[0, 0] = jnp.where(cha, a, bb)
        ok_ref[0, 1] = jnp.where(cha, bb, a)
        ow_ref[0, 0] = jnp.where(cha, wa, wb)
        ow_ref[0, 1] = jnp.where(cha, wb, wa)
    return _pair_body


def _bitonic_sort(key2d, w2d):
    blk = pl.BlockSpec((RB, 128), lambda i: (i, I32(0)))
    shp = [jax.ShapeDtypeStruct((SPAD // 128, 128), I32),
           jax.ShapeDtypeStruct((SPAD // 128, 128), F32)]
    key2d, w2d = pl.pallas_call(
        _local_sort_body, grid=(NBLK,),
        in_specs=[blk, blk], out_specs=[blk, blk], out_shape=shp,
    )(key2d, w2d)
    nrows = SPAD // 128
    for k in range(LOGM + 1, LOGN + 1):
        for j in range(k - 1, LOGM - 1, -1):
            db = 1 << (j - LOGM)
            half = db * RB
            G = nrows // (2 * half)
            k4 = key2d.reshape(G, 2, half, 128)
            w4 = w2d.reshape(G, 2, half, 128)
            pblk = pl.BlockSpec((1, 2, RB, 128), lambda g, i: (g, I32(0), i, I32(0)))
            k4, w4 = pl.pallas_call(
                _pair_body_factory(k, j), grid=(G, db),
                in_specs=[pblk, pblk], out_specs=[pblk, pblk],
                out_shape=[jax.ShapeDtypeStruct((G, 2, half, 128), I32),
                           jax.ShapeDtypeStruct((G, 2, half, 128), F32)],
            )(k4, w4)
            key2d = k4.reshape(nrows, 128)
            w2d = w4.reshape(nrows, 128)
        key2d, w2d = pl.pallas_call(
            _finish_body_factory(k), grid=(NBLK,),
            in_specs=[blk, blk], out_specs=[blk, blk], out_shape=shp,
        )(key2d, w2d)
    return key2d, w2d


def kernel(gradient, block_edge_index, edge_index, edge_weight, step_size):
    gpad = jnp.concatenate(
        [gradient, jnp.full((GPAD - N_GRAD,), -jnp.inf, F32)]).reshape(1024, 1024)
    keys2d, npos_part = _make_keys(gpad)
    key = keys2d.reshape(GPAD)
    n_pos = jnp.sum(npos_part).astype(jnp.int64)

    topv, topi = lax.top_k(gradient, STEP_SIZE)

    flip_ei = block_edge_index[:, topi]
    fsrc = flip_ei[0].astype(jnp.uint32)
    fdst = flip_ei[1].astype(jnp.uint32)
    fw = jnp.asarray(step_size // STEP_SIZE).astype(F32)
    esrc = edge_index[0].astype(jnp.uint32)
    edst = edge_index[1].astype(jnp.uint32)
    NN = jnp.uint32(NUM_NODES)
    ids = jnp.concatenate([esrc * NN + edst, fsrc * NN + fdst, fdst * NN + fsrc])
    w = jnp.concatenate([edge_weight.astype(F32),
                         jnp.full((2 * STEP_SIZE,), 1.0, F32) * fw])

    skey = (ids ^ jnp.uint32(0x80000000)).astype(I32)
    skey_p = jnp.concatenate(
        [skey, jnp.full((SPAD - TOT,), I32(0x7FFFFFFF))]).reshape(SPAD // 128, 128)
    w_p = jnp.concatenate(
        [w, jnp.zeros((SPAD - TOT,), F32)]).reshape(SPAD // 128, 128)
    skey_s2d, w_s2d = _bitonic_sort(skey_p, w_p)
    ids_s = (skey_s2d.reshape(SPAD).astype(jnp.uint32)
             ^ jnp.uint32(0x80000000))[:TOT]
    w_s = w_s2d.reshape(SPAD)[:TOT]

    newrun = jnp.concatenate([jnp.array([True]), ids_s[1:] != ids_s[:-1]])
    S = jnp.cumsum(w_s)
    run_id = jnp.cumsum(newrun.astype(I32)) - I32(1)
    nruns = run_id[-1] + I32(1)
    pos = jnp.arange(TOT, dtype=I32)
    run_starts = jnp.full((TOT,), TOT, I32).at[
        jnp.where(newrun, run_id, I32(TOT))].min(pos, mode='drop')
    valid = pos < nruns
    rs = jnp.where(valid, run_starts, I32(0))
    next_start = jnp.where(pos + I32(1) < nruns, jnp.roll(run_starts, -1),
                           I32(TOT))
    end_idx = jnp.where(valid, next_start - I32(1), I32(0))
    summed = jnp.where(valid, S[end_idx] - jnp.where(rs > I32(0),
                                                     S[rs - I32(1)], 0.0),
                       0.0).astype(F32)
    id_k = ids_s[rs]
    PAD = jnp.iinfo(jnp.int64).min
    rep_src = jnp.where(valid, (id_k // NUM_NODES).astype(jnp.int64), PAD)
    rep_dst = jnp.where(valid, (id_k % NUM_NODES).astype(jnp.int64), PAD)
    final_w = jnp.where(jnp.isclose(summed, 1.0), summed, 0.0)
    return final_w, rep_src, rep_dst, topv, flip_ei, n_pos
